# Initial kernel scaffold; baseline (speedup 1.0000x reference)
import jax
import jax.numpy as jnp
from jax.experimental import pallas as pl


def kernel(tokens, table):
    raise NotImplementedError("write your pallas kernel here")


# speedup vs baseline: 1.3137x; 1.3137x over previous
import functools
import math

import jax
import jax.numpy as jnp
from jax import lax
from jax.experimental import pallas as pl
from jax.experimental.pallas import tpu as pltpu
from jax.experimental.pallas import tpu_sc as plsc

VOCAB = 1000000
EMB = 32
SCALE = math.sqrt(EMB)

NC = 2
NS = 16
NW = NC * NS
LANES = 16

B_TOTAL = 4096 * 200
B_PER_W = B_TOTAL // NW
CHUNK = 1600
NCHUNK = B_PER_W // CHUNK


@functools.partial(
    pl.kernel,
    out_type=jax.ShapeDtypeStruct((B_TOTAL, EMB), jnp.float32),
    mesh=plsc.VectorSubcoreMesh(core_axis_name="c", subcore_axis_name="s"),
    scratch_types=[
        pltpu.VMEM((B_PER_W,), jnp.int32),
        pltpu.VMEM((CHUNK, EMB), jnp.float32),
        pltpu.SemaphoreType.DMA,
    ],
    compiler_params=pltpu.CompilerParams(use_tc_tiling_on_sc=False),
)
def _emb_lookup(table_hbm, idx_hbm, out_hbm, idx_v, rows_v, sem):
    wid = lax.axis_index("s") * NC + lax.axis_index("c")
    base = wid * B_PER_W
    pltpu.sync_copy(idx_hbm.at[pl.ds(base, B_PER_W)], idx_v)

    def scale_body(r, carry):
        for j in range(EMB // LANES):
            sl = pl.ds(j * LANES, LANES)
            rows_v[r, sl] = rows_v[r, sl] * SCALE
        return carry

    for c in range(NCHUNK):
        off = c * CHUNK
        pltpu.async_copy(
            table_hbm.at[idx_v.at[pl.ds(off, CHUNK)]], rows_v, sem
        ).wait()
        lax.fori_loop(0, CHUNK, scale_body, 0)
        pltpu.sync_copy(rows_v, out_hbm.at[pl.ds(base + off, CHUNK)])


def kernel(tokens, table):
    b, s = tokens.shape
    idx = tokens.reshape(-1).astype(jnp.int32)
    out = _emb_lookup(table, idx)
    return out.reshape(b, s, EMB)


# speedup vs baseline: 1.4762x; 1.1236x over previous
import functools
import math

import jax
import jax.numpy as jnp
from jax import lax
from jax.experimental import pallas as pl
from jax.experimental.pallas import tpu as pltpu
from jax.experimental.pallas import tpu_sc as plsc

VOCAB = 1000000
EMB = 32
SCALE = math.sqrt(EMB)

NC = 2
NS = 16
NW = NC * NS
LANES = 16

B_TOTAL = 4096 * 200
B_PER_W = B_TOTAL // NW
CHUNK = 1600
NCHUNK = B_PER_W // CHUNK


@functools.partial(
    pl.kernel,
    out_type=jax.ShapeDtypeStruct((B_TOTAL, EMB), jnp.float32),
    mesh=plsc.VectorSubcoreMesh(core_axis_name="c", subcore_axis_name="s"),
    scratch_types=[
        pltpu.VMEM((B_PER_W,), jnp.int32),
        pltpu.VMEM((CHUNK, EMB), jnp.float32),
        pltpu.VMEM((CHUNK, EMB), jnp.float32),
        pltpu.SemaphoreType.DMA,
        pltpu.SemaphoreType.DMA,
        pltpu.SemaphoreType.DMA,
        pltpu.SemaphoreType.DMA,
    ],
    compiler_params=pltpu.CompilerParams(use_tc_tiling_on_sc=False),
)
def _emb_lookup(table_hbm, idx_hbm, out_hbm, idx_v, rows0, rows1,
                gsem0, gsem1, osem0, osem1):
    wid = lax.axis_index("s") * NC + lax.axis_index("c")
    base = wid * B_PER_W
    pltpu.sync_copy(idx_hbm.at[pl.ds(base, B_PER_W)], idx_v)

    bufs = (rows0, rows1)
    gsems = (gsem0, gsem1)
    osems = (osem0, osem1)

    def start_gather(c):
        b = c % 2
        return pltpu.async_copy(
            table_hbm.at[idx_v.at[pl.ds(c * CHUNK, CHUNK)]], bufs[b], gsems[b]
        )

    def scale(buf):
        @plsc.parallel_loop(0, CHUNK, unroll=8)
        def _(r):
            for j in range(EMB // LANES):
                sl = pl.ds(j * LANES, LANES)
                buf[r, sl] = buf[r, sl] * SCALE

    gh = [start_gather(0), None]
    oh = [None, None]
    for c in range(NCHUNK):
        b = c % 2
        if c + 1 < NCHUNK:
            if oh[1 - b] is not None:
                oh[1 - b].wait()
                oh[1 - b] = None
            gh[1 - b] = start_gather(c + 1)
        gh[b].wait()
        scale(bufs[b])
        oh[b] = pltpu.async_copy(
            bufs[b], out_hbm.at[pl.ds(base + c * CHUNK, CHUNK)], osems[b]
        )
    for b in range(2):
        if oh[b] is not None:
            oh[b].wait()


def kernel(tokens, table):
    b, s = tokens.shape
    idx = tokens.reshape(-1).astype(jnp.int32)
    out = _emb_lookup(table, idx)
    return out.reshape(b, s, EMB)
